# baseline (speedup 1.0000x reference)
import functools
import jax
import jax.numpy as jnp
from jax import lax
from jax.experimental import pallas as pl
from jax.experimental.pallas import tpu as pltpu
from jax.experimental.pallas import tpu_sc as plsc

_B, _F, _D = 4096, 100, 64
_K = 50
_NW = 32
_B_PER_W = _B // _NW
_NB = 4
_NCHUNK = _B_PER_W // _NB


@functools.partial(
    pl.kernel,
    mesh=plsc.VectorSubcoreMesh(core_axis_name="c", subcore_axis_name="s"),
    out_type=jax.ShapeDtypeStruct((_B, _K, _D), jnp.float32),
    scratch_types=[
        pltpu.VMEM((_NB, _F, _D), jnp.float32),
        pltpu.VMEM((_NB, _K, _D), jnp.float32),
        pltpu.SemaphoreType.DMA,
    ],
)
def _masked_compact(in_hbm, out_hbm, in_v, out_v, sem):
    wid = lax.axis_index("s") * 2 + lax.axis_index("c")
    base = wid * _B_PER_W

    def chunk(g, carry):
        b0 = base + g * _NB
        pltpu.async_copy(in_hbm.at[pl.ds(b0, _NB), :, :], in_v, sem).wait()

        def row(r, c):
            b = r // _K
            j = r % _K
            for k in range(4):
                out_v[b, j, pl.ds(k * 16, 16)] = in_v[b, 2 * j, pl.ds(k * 16, 16)]
            return c

        lax.fori_loop(0, _NB * _K, row, 0)
        pltpu.sync_copy(out_v, out_hbm.at[pl.ds(b0, _NB), :, :])
        return carry

    lax.fori_loop(0, _NCHUNK, chunk, 0)


def kernel(inputs):
    return _masked_compact(inputs)


# speedup vs baseline: 6.9214x; 6.9214x over previous
import functools
import jax
import jax.numpy as jnp
from jax import lax
from jax.experimental import pallas as pl
from jax.experimental.pallas import tpu as pltpu
from jax.experimental.pallas import tpu_sc as plsc

_B, _F, _D = 4096, 100, 64
_K = 50
_SB = 4
_NU = 25


@functools.partial(
    pl.kernel,
    mesh=plsc.VectorSubcoreMesh(core_axis_name="c", subcore_axis_name="s"),
    out_type=jax.ShapeDtypeStruct((_K, _D, _B), jnp.float32),
    scratch_types=[
        pltpu.VMEM((_SB, _B), jnp.float32),
        pltpu.VMEM((_SB, _B), jnp.float32),
        pltpu.SemaphoreType.DMA,
        pltpu.SemaphoreType.DMA,
        pltpu.SemaphoreType.DMA,
        pltpu.SemaphoreType.DMA,
    ],
)
def _masked_compact(in_hbm, out_hbm, buf0, buf1, si0, si1, so0, so1):
    wid = lax.axis_index("s") * 2 + lax.axis_index("c")
    bi = _SB * (wid % 16)
    p = wid // 16

    bufs = (buf0, buf1)
    isems = (si0, si1)
    osems = (so0, so1)

    def start_in(u):
        k = u % 2
        return pltpu.async_copy(
            in_hbm.at[2 * p + 4 * u, pl.ds(bi, _SB), :], bufs[k], isems[k]
        )

    def start_out(u):
        k = u % 2
        return pltpu.async_copy(
            bufs[k], out_hbm.at[p + 2 * u, pl.ds(bi, _SB), :], osems[k]
        )

    ins = {0: start_in(0)}
    outs = {}
    for u in range(_NU):
        k = u % 2
        ins[u].wait()
        if u + 1 < _NU:
            if u - 1 >= 0:
                outs[u - 1].wait()
            ins[u + 1] = start_in(u + 1)
        outs[u] = start_out(u)
    outs[_NU - 2].wait()
    outs[_NU - 1].wait()


def kernel(inputs):
    xt = jnp.transpose(inputs, (1, 2, 0))
    yt = _masked_compact(xt)
    return jnp.transpose(yt, (2, 0, 1))
